# Initial kernel scaffold; baseline (speedup 1.0000x reference)
import jax
import jax.numpy as jnp
from jax.experimental import pallas as pl


def kernel(input, frozen_weight, raw_weight):
    raise NotImplementedError("write your pallas kernel here")


# speedup vs baseline: 1.3565x; 1.3565x over previous
import functools

import jax
import jax.numpy as jnp
from jax import lax
from jax.experimental import pallas as pl
from jax.experimental.pallas import tpu as pltpu
from jax.experimental.pallas import tpu_sc as plsc

VOCAB = 1000000
EXTRA = 10000
DIM = 32

NC = 2
NSUB = 16
L = 16
NW = NC * NSUB

N = 4096 * 200
PER_W = N // NW
CHUNK = 1024
NCHUNK = PER_W // CHUNK
SUBDMA = 128
NDMA = CHUNK // SUBDMA


def _emb_body(idx_hbm, frozen_hbm, raw_hbm, out_hbm,
              idx_v, clamp_v, ridx_v, rpos_v, rows_v, stage_v, sem, semp):
    wid = lax.axis_index("s") * NC + lax.axis_index("c")
    wbase = wid * PER_W

    def chunk_body(ci, carry):
        gbase = wbase + ci * CHUNK
        pltpu.sync_copy(idx_hbm.at[pl.ds(gbase, CHUNK)], idx_v)

        vocab_s = jnp.full((L,), VOCAB, jnp.int32)
        one_s = jnp.full((L,), 1, jnp.int32)
        iota = lax.iota(jnp.int32, L)
        cursor = jnp.int32(0)
        for i in range(CHUNK // L):
            iv = idx_v[pl.ds(i * L, L)]
            m = iv >= vocab_s
            vpr = SUBDMA // L
            clamp_v[i // vpr, pl.ds((i % vpr) * L, L)] = jnp.where(
                m, vocab_s - one_s, iv)
            mi = jnp.where(m, one_s, one_s - one_s)
            incl = plsc.cumsum(mi)
            cnt = incl[L - 1]

            @pl.when(cnt > 0)
            def _():
                pos = jnp.full((L,), cursor, jnp.int32) + incl - one_s
                plsc.store_scatter(ridx_v, [pos], iv - vocab_s, mask=m)
                gpos = jnp.full((L,), gbase + i * L, jnp.int32) + iota
                plsc.store_scatter(rpos_v, [pos], gpos, mask=m)

            cursor = cursor + cnt

        copies = [
            pltpu.async_copy(
                frozen_hbm.at[clamp_v.at[j]],
                rows_v.at[pl.ds(j * SUBDMA, SUBDMA)],
                sem,
            )
            for j in range(NDMA)
        ]
        for c in copies:
            c.wait()
        pltpu.sync_copy(rows_v, out_hbm.at[pl.ds(gbase, CHUNK)])

        n = cursor

        @pl.when(n > 0)
        def _():
            head_r = ridx_v[pl.ds(0, L)]
            head_g = rpos_v[pl.ds(0, L)]
            r0 = jnp.full((L,), head_r[0], jnp.int32)
            g0 = jnp.full((L,), head_g[0], jnp.int32)
            n_s = jnp.full((L,), n, jnp.int32)

            def patch(g, carry2):
                lanes = jnp.full((L,), g * L, jnp.int32) + iota
                valid = lanes < n_s
                rv = jnp.where(valid, plsc.load_gather(ridx_v, [lanes]), r0)
                gv = jnp.where(valid, plsc.load_gather(rpos_v, [lanes]), g0)
                pltpu.async_copy(raw_hbm.at[rv], stage_v, semp).wait()
                pltpu.async_copy(stage_v, out_hbm.at[gv], semp).wait()
                return carry2

            lax.fori_loop(0, (n + L - 1) // L, patch, 0)

        return carry

    lax.fori_loop(0, NCHUNK, chunk_body, 0)


@functools.partial(jax.jit, static_argnums=())
def _emb(idx, frozen_weight, raw_weight):
    mesh = plsc.VectorSubcoreMesh(core_axis_name="c", subcore_axis_name="s")
    run = functools.partial(
        pl.kernel,
        out_type=jax.ShapeDtypeStruct((N, DIM), jnp.float32),
        mesh=mesh,
        compiler_params=pltpu.CompilerParams(
            needs_layout_passes=False, use_tc_tiling_on_sc=False),
        scratch_types=[
            pltpu.VMEM((CHUNK,), jnp.int32),
            pltpu.VMEM((NDMA, SUBDMA), jnp.int32),
            pltpu.VMEM((CHUNK + L,), jnp.int32),
            pltpu.VMEM((CHUNK + L,), jnp.int32),
            pltpu.VMEM((CHUNK, DIM), jnp.float32),
            pltpu.VMEM((L, DIM), jnp.float32),
            pltpu.SemaphoreType.DMA,
            pltpu.SemaphoreType.DMA,
        ],
    )(_emb_body)
    return run(idx, frozen_weight, raw_weight)


def kernel(input, frozen_weight, raw_weight):
    idx = input.reshape(-1).astype(jnp.int32)
    out = _emb(idx, frozen_weight, raw_weight)
    return out.reshape(input.shape + (DIM,))


# speedup vs baseline: 1.3627x; 1.0046x over previous
import functools

import jax
import jax.numpy as jnp
from jax import lax
from jax.experimental import pallas as pl
from jax.experimental.pallas import tpu as pltpu
from jax.experimental.pallas import tpu_sc as plsc

VOCAB = 1000000
EXTRA = 10000
DIM = 32

NC = 2
NSUB = 16
L = 16
NW = NC * NSUB

N = 4096 * 200
PER_W = N // NW
SUBDMA = 128
CHUNK = 1024
NDMA = CHUNK // SUBDMA
NCHUNK = PER_W // CHUNK
VPR = SUBDMA // L
NROW = N // SUBDMA
WROW = PER_W // SUBDMA


def _emb_body(idx_hbm, frozen_hbm, raw_hbm, out_hbm,
              idx0, idx1, ridx0, ridx1, rpos0, rpos1,
              rows_v, stage_v, semg, semi, semp):
    wid = lax.axis_index("s") * NC + lax.axis_index("c")
    wrow = wid * WROW
    iota = lax.iota(jnp.int32, L)
    vocab_s = jnp.full((L,), VOCAB, jnp.int32)
    one_s = jnp.full((L,), 1, jnp.int32)

    idx_b = (idx0, idx1)
    ridx_b = (ridx0, ridx1)
    rpos_b = (rpos0, rpos1)

    def idx_copy(c, p):
        return pltpu.make_async_copy(
            idx_hbm.at[pl.ds(wrow + c * NDMA, NDMA)], idx_b[p], semi)

    def load_idx(c, p):
        idx_copy(c, p).start()

    def drain_idx(p):
        idx_copy(0, p).wait()

    def compact(c, p):
        idx_v, ridx_v, rpos_v = idx_b[p], ridx_b[p], rpos_b[p]
        gbase = (wrow + c * NDMA) * SUBDMA
        cursor = jnp.int32(0)
        for i in range(CHUNK // L):
            r, o = i // VPR, (i % VPR) * L
            iv = idx_v[r, pl.ds(o, L)]
            m = iv >= vocab_s
            idx_v[r, pl.ds(o, L)] = jnp.where(m, vocab_s - one_s, iv)
            cnt = plsc.all_reduce_population_count(m)[0]

            @pl.when(cnt > 0)
            def _():
                incl = plsc.cumsum(jnp.where(m, one_s, one_s - one_s))
                pos = jnp.full((L,), cursor, jnp.int32) + incl - one_s
                plsc.store_scatter(ridx_v, [pos], iv - vocab_s, mask=m)
                gpos = jnp.full((L,), gbase + i * L, jnp.int32) + iota
                plsc.store_scatter(rpos_v, [pos], gpos, mask=m)

            cursor = cursor + cnt
        return cursor

    def gather_copy(p, j):
        return pltpu.make_async_copy(
            frozen_hbm.at[idx_b[p].at[j]],
            rows_v.at[pl.ds(j * SUBDMA, SUBDMA)],
            semg,
        )

    def fire_gathers(p):
        for j in range(NDMA):
            gather_copy(p, j).start()

    def drain_gathers(p):
        for j in range(NDMA):
            gather_copy(p, j).wait()

    def store_rows(c):
        pltpu.sync_copy(rows_v, out_hbm.at[pl.ds((wrow + c * NDMA) * SUBDMA,
                                                 CHUNK)])

    def patch(c, p, n):
        ridx_v, rpos_v = ridx_b[p], rpos_b[p]

        @pl.when(n > 0)
        def _():
            r0 = jnp.full((L,), ridx_v[pl.ds(0, L)][0], jnp.int32)
            g0 = jnp.full((L,), rpos_v[pl.ds(0, L)][0], jnp.int32)
            n_s = jnp.full((L,), n, jnp.int32)

            def step(g, carry2):
                lanes = jnp.full((L,), g * L, jnp.int32) + iota
                valid = lanes < n_s
                rv = jnp.where(valid, plsc.load_gather(ridx_v, [lanes]), r0)
                gv = jnp.where(valid, plsc.load_gather(rpos_v, [lanes]), g0)
                pltpu.async_copy(raw_hbm.at[rv], stage_v, semp).wait()
                pltpu.async_copy(stage_v, out_hbm.at[gv], semp).wait()
                return carry2

            lax.fori_loop(0, (n + L - 1) // L, step, 0)

    def half(c, p, q, n_cur):
        drain_idx(q)
        n_next = compact(c + 1, q)
        drain_gathers(p)
        store_rows(c)
        fire_gathers(q)
        load_idx(jnp.minimum(c + 2, NCHUNK - 1), p)
        patch(c, p, n_cur)
        return n_next

    load_idx(0, 0)
    drain_idx(0)
    n_cur = compact(0, 0)
    fire_gathers(0)
    load_idx(1, 1)

    def pair_body(g, n_in):
        n_mid = half(2 * g, 0, 1, n_in)
        return half(2 * g + 1, 1, 0, n_mid)

    n_cur = lax.fori_loop(0, (NCHUNK - 1) // 2, pair_body, n_cur)

    drain_idx(1)
    drain_gathers(0)
    store_rows(NCHUNK - 1)
    patch(NCHUNK - 1, 0, n_cur)


@jax.jit
def _emb(idx2d, frozen_weight, raw_weight):
    mesh = plsc.VectorSubcoreMesh(core_axis_name="c", subcore_axis_name="s")
    run = functools.partial(
        pl.kernel,
        out_type=jax.ShapeDtypeStruct((N, DIM), jnp.float32),
        mesh=mesh,
        compiler_params=pltpu.CompilerParams(
            needs_layout_passes=False, use_tc_tiling_on_sc=False),
        scratch_types=[
            pltpu.VMEM((NDMA, SUBDMA), jnp.int32),
            pltpu.VMEM((NDMA, SUBDMA), jnp.int32),
            pltpu.VMEM((CHUNK + L,), jnp.int32),
            pltpu.VMEM((CHUNK + L,), jnp.int32),
            pltpu.VMEM((CHUNK + L,), jnp.int32),
            pltpu.VMEM((CHUNK + L,), jnp.int32),
            pltpu.VMEM((CHUNK, DIM), jnp.float32),
            pltpu.VMEM((L, DIM), jnp.float32),
            pltpu.SemaphoreType.DMA,
            pltpu.SemaphoreType.DMA,
            pltpu.SemaphoreType.DMA,
        ],
    )(_emb_body)
    return run(idx2d, frozen_weight, raw_weight)


def kernel(input, frozen_weight, raw_weight):
    idx2d = input.reshape(NROW, SUBDMA).astype(jnp.int32)
    out = _emb(idx2d, frozen_weight, raw_weight)
    return out.reshape(input.shape + (DIM,))
